# Initial kernel scaffold; baseline (speedup 1.0000x reference)
import jax
import jax.numpy as jnp
from jax.experimental import pallas as pl


def kernel(x, edge_index, edge_attr, ground_node, node_subnode_index, subgraph_edge_index, subnode_node_index, batch, params):
    raise NotImplementedError("write your pallas kernel here")


# speedup vs baseline: 12.4211x; 12.4211x over previous
import functools

import jax
import jax.numpy as jnp
from jax import lax
from jax.experimental import pallas as pl
from jax.experimental.pallas import tpu as pltpu
from jax.experimental.pallas import tpu_sc as plsc

_N = 10000
_E = 160000
_D = 256
_H = 256
_HH = 128
_OUT = 128
_G = 64
_NS = 16
_CH = 80
_NCH = _E // _NS // _CH
_EPS = _E // _NS
_RPS = 624
_RPS_LAST = _N - 15 * _RPS
_BLK = 1000
_NBLK = _N // _BLK

_f32 = jnp.float32


def _sc_mesh():
    return plsc.VectorSubcoreMesh(core_axis_name="c", subcore_axis_name="s")


def _deg_sc(zeros_n, d0, d1, d2, d3):
    @functools.partial(
        pl.kernel,
        out_type=[jax.ShapeDtypeStruct((_N,), _f32) for _ in range(4)],
        mesh=_sc_mesh(),
        scratch_types=[
            pltpu.VMEM_SHARED((_N,), _f32),
            pltpu.VMEM_SHARED((_N,), _f32),
            pltpu.VMEM((_NCH, _CH), jnp.int32),
            pltpu.VMEM((_CH,), _f32),
        ],
    )
    def k(z_hbm, d0_hbm, d1_hbm, d2_hbm, d3_hbm,
          o0_hbm, o1_hbm, o2_hbm, o3_hbm,
          degA_sh, degB_sh, slab_v, ones_v):
        cid = lax.axis_index("c")
        sid = lax.axis_index("s")

        @pl.when(sid == 0)
        def _():
            pltpu.sync_copy(z_hbm, degA_sh)

        @pl.when(sid == 1)
        def _():
            pltpu.sync_copy(z_hbm, degB_sh)

        for i in range(_CH // 16):
            ones_v[pl.ds(i * 16, 16)] = jnp.ones((16,), _f32)
        plsc.subcore_barrier()

        dsts = (d0_hbm, d1_hbm, d2_hbm, d3_hbm)
        outs = (o0_hbm, o1_hbm, o2_hbm, o3_hbm)
        for c in range(2):
            @pl.when(cid == c)
            def _(c=c):
                for local, acc_sh in ((0, degA_sh), (1, degB_sh)):
                    pltpu.sync_copy(dsts[2 * c + local].at[sid], slab_v)

                    def body(j, _, acc_sh=acc_sh):
                        pltpu.sync_copy(ones_v, acc_sh.at[slab_v.at[j]],
                                        add=True)
                        return 0

                    lax.fori_loop(0, _NCH, body, 0)
        plsc.subcore_barrier()

        for c in range(2):
            @pl.when(cid == c)
            def _(c=c):
                @pl.when(sid == 0)
                def _():
                    pltpu.sync_copy(degA_sh, outs[2 * c])

                @pl.when(sid == 1)
                def _():
                    pltpu.sync_copy(degB_sh, outs[2 * c + 1])

    return k(zeros_n, d0, d1, d2, d3)


@functools.partial(
    pl.kernel,
    out_type=[jax.ShapeDtypeStruct((_N, _HH), _f32),
              jax.ShapeDtypeStruct((_N, _HH), _f32)],
    mesh=_sc_mesh(),
    scratch_types=[
        pltpu.VMEM_SHARED((_N, _HH), _f32),
        pltpu.VMEM((_NCH, _CH), jnp.int32),
        pltpu.VMEM((_CH,), jnp.int32),
        pltpu.VMEM((_CH,), jnp.int32),
        pltpu.VMEM((_CH, _HH), _f32),
        pltpu.VMEM((_CH, _HH), _f32),
        pltpu.SemaphoreType.DMA,
        pltpu.SemaphoreType.DMA,
    ],
)
def _conv_sc(gl_hbm, gr_hbm, src_hbm, dst_hbm, ol_hbm, or_hbm,
             acc_sh, dst_v, src0_v, src1_v, rows0_v, rows1_v, sem0, sem1):
    cid = lax.axis_index("c")
    sid = lax.axis_index("s")

    pltpu.sync_copy(dst_hbm.at[sid], dst_v)
    ebase = sid * _EPS

    for c, g_hbm, o_hbm in ((0, gl_hbm, ol_hbm), (1, gr_hbm, or_hbm)):
        @pl.when(cid == c)
        def _(g_hbm=g_hbm, o_hbm=o_hbm):
            @pl.when(sid < 15)
            def _():
                pltpu.sync_copy(g_hbm.at[pl.ds(sid * _RPS, _RPS)],
                                acc_sh.at[pl.ds(sid * _RPS, _RPS)])

            @pl.when(sid == 15)
            def _():
                pltpu.sync_copy(g_hbm.at[pl.ds(15 * _RPS, _RPS_LAST)],
                                acc_sh.at[pl.ds(15 * _RPS, _RPS_LAST)])
            plsc.subcore_barrier()

            def pair(i, _):
                j0 = 2 * i
                j1 = 2 * i + 1
                pltpu.sync_copy(src_hbm.at[pl.ds(ebase + j0 * _CH, _CH)],
                                src0_v)
                a = pltpu.async_copy(g_hbm.at[src0_v], rows0_v, sem0)
                pltpu.sync_copy(src_hbm.at[pl.ds(ebase + j1 * _CH, _CH)],
                                src1_v)
                b = pltpu.async_copy(g_hbm.at[src1_v], rows1_v, sem1)
                a.wait()
                pltpu.sync_copy(rows0_v, acc_sh.at[dst_v.at[j0]], add=True)
                b.wait()
                pltpu.sync_copy(rows1_v, acc_sh.at[dst_v.at[j1]], add=True)
                return 0

            lax.fori_loop(0, _NCH // 2, pair, 0)
            jt = _NCH - 1
            pltpu.sync_copy(src_hbm.at[pl.ds(ebase + jt * _CH, _CH)], src0_v)
            pltpu.async_copy(g_hbm.at[src0_v], rows0_v, sem0).wait()
            pltpu.sync_copy(rows0_v, acc_sh.at[dst_v.at[jt]], add=True)

            plsc.subcore_barrier()

            @pl.when(sid < 15)
            def _():
                pltpu.sync_copy(acc_sh.at[pl.ds(sid * _RPS, _RPS)],
                                o_hbm.at[pl.ds(sid * _RPS, _RPS)])

            @pl.when(sid == 15)
            def _():
                pltpu.sync_copy(acc_sh.at[pl.ds(15 * _RPS, _RPS_LAST)],
                                o_hbm.at[pl.ds(15 * _RPS, _RPS_LAST)])


def _dot(a, b):
    return jnp.dot(a, b, preferred_element_type=_f32)


def _embed_tc(x, we, be, w1, deg1):
    def body(x_ref, we_ref, be_ref, w1_ref, deg_ref, h0_ref, gl_ref, gr_ref):
        h0 = _dot(x_ref[...], we_ref[...]) + be_ref[...]
        h0_ref[...] = h0
        dinv = lax.rsqrt(deg_ref[...] + 1.0)
        g = _dot(h0, w1_ref[...]) * dinv
        gl_ref[...] = g[:, :_HH]
        gr_ref[...] = g[:, _HH:]

    return pl.pallas_call(
        body,
        grid=(_NBLK,),
        in_specs=[
            pl.BlockSpec((_BLK, _D), lambda i: (i, 0)),
            pl.BlockSpec((_D, _H), lambda i: (0, 0)),
            pl.BlockSpec((1, _H), lambda i: (0, 0)),
            pl.BlockSpec((_H, _H), lambda i: (0, 0)),
            pl.BlockSpec((_BLK, 1), lambda i: (i, 0)),
        ],
        out_specs=[
            pl.BlockSpec((_BLK, _H), lambda i: (i, 0)),
            pl.BlockSpec((_BLK, _HH), lambda i: (i, 0)),
            pl.BlockSpec((_BLK, _HH), lambda i: (i, 0)),
        ],
        out_shape=[
            jax.ShapeDtypeStruct((_N, _H), _f32),
            jax.ShapeDtypeStruct((_N, _HH), _f32),
            jax.ShapeDtypeStruct((_N, _HH), _f32),
        ],
    )(x, we, be, w1, deg1)


def _merge_tc(al, ar, h, deg, b, m, wn, degn):
    def body(al_ref, ar_ref, h_ref, deg_ref, b_ref, m_ref, wn_ref, degn_ref,
             ho_ref, gl_ref, gr_ref):
        dinv = lax.rsqrt(deg_ref[...] + 1.0)
        acc = jnp.concatenate([al_ref[...], ar_ref[...]], axis=-1) * dinv
        hn = jnp.where(m_ref[...] == 1.0,
                       jnp.maximum(acc + b_ref[...], 0.0), h_ref[...])
        ho_ref[...] = hn
        g = _dot(hn, wn_ref[...]) * lax.rsqrt(degn_ref[...] + 1.0)
        gl_ref[...] = g[:, :_HH]
        gr_ref[...] = g[:, _HH:]

    return pl.pallas_call(
        body,
        grid=(_NBLK,),
        in_specs=[
            pl.BlockSpec((_BLK, _HH), lambda i: (i, 0)),
            pl.BlockSpec((_BLK, _HH), lambda i: (i, 0)),
            pl.BlockSpec((_BLK, _H), lambda i: (i, 0)),
            pl.BlockSpec((_BLK, 1), lambda i: (i, 0)),
            pl.BlockSpec((1, _H), lambda i: (0, 0)),
            pl.BlockSpec((_BLK, 1), lambda i: (i, 0)),
            pl.BlockSpec((_H, _H), lambda i: (0, 0)),
            pl.BlockSpec((_BLK, 1), lambda i: (i, 0)),
        ],
        out_specs=[
            pl.BlockSpec((_BLK, _H), lambda i: (i, 0)),
            pl.BlockSpec((_BLK, _HH), lambda i: (i, 0)),
            pl.BlockSpec((_BLK, _HH), lambda i: (i, 0)),
        ],
        out_shape=[
            jax.ShapeDtypeStruct((_N, _H), _f32),
            jax.ShapeDtypeStruct((_N, _HH), _f32),
            jax.ShapeDtypeStruct((_N, _HH), _f32),
        ],
    )(al, ar, h, deg, b, m, wn, degn)


def _final_tc(al, ar, h, deg, b, m, batch2, wh, bh):
    def body(al_ref, ar_ref, h_ref, deg_ref, b_ref, m_ref, batch_ref,
             wh_ref, bh_ref, out_ref, pooled):
        i = pl.program_id(0)
        dinv = lax.rsqrt(deg_ref[...] + 1.0)
        acc = jnp.concatenate([al_ref[...], ar_ref[...]], axis=-1) * dinv
        hn = jnp.where(m_ref[...] == 1.0,
                       jnp.maximum(acc + b_ref[...], 0.0), h_ref[...])
        seg = lax.broadcasted_iota(jnp.int32, (1, _G), 1)
        onehot = (batch_ref[...] == seg).astype(_f32)
        contrib = lax.dot_general(onehot, hn, (((0,), (0,)), ((), ())),
                                  preferred_element_type=_f32)

        @pl.when(i == 0)
        def _():
            pooled[...] = contrib

        @pl.when(i > 0)
        def _():
            pooled[...] += contrib

        @pl.when(i == _NBLK - 1)
        def _():
            out_ref[...] = _dot(pooled[...], wh_ref[...]) + bh_ref[...]

    return pl.pallas_call(
        body,
        grid=(_NBLK,),
        in_specs=[
            pl.BlockSpec((_BLK, _HH), lambda i: (i, 0)),
            pl.BlockSpec((_BLK, _HH), lambda i: (i, 0)),
            pl.BlockSpec((_BLK, _H), lambda i: (i, 0)),
            pl.BlockSpec((_BLK, 1), lambda i: (i, 0)),
            pl.BlockSpec((1, _H), lambda i: (0, 0)),
            pl.BlockSpec((_BLK, 1), lambda i: (i, 0)),
            pl.BlockSpec((_BLK, 1), lambda i: (i, 0)),
            pl.BlockSpec((_H, _OUT), lambda i: (0, 0)),
            pl.BlockSpec((1, _OUT), lambda i: (0, 0)),
        ],
        out_specs=pl.BlockSpec((_G, _OUT), lambda i: (0, 0)),
        out_shape=jax.ShapeDtypeStruct((_G, _OUT), _f32),
        scratch_shapes=[pltpu.VMEM((_G, _H), _f32)],
        compiler_params=pltpu.CompilerParams(
            dimension_semantics=("arbitrary",)),
    )(al, ar, h, deg, b, m, batch2, wh, bh)


def kernel(x, edge_index, edge_attr, ground_node, node_subnode_index,
           subgraph_edge_index, subnode_node_index, batch, params):
    sets = (edge_index, node_subnode_index, subgraph_edge_index,
            subnode_node_index)
    src1 = [s[0] for s in sets]
    dst3 = [s[1].reshape(_NS, _NCH, _CH) for s in sets]

    degs = _deg_sc(jnp.zeros((_N,), _f32),
                   dst3[0], dst3[1], dst3[2], dst3[3])
    degc = [d.reshape(_N, 1) for d in degs]

    gnf = ground_node.astype(_f32).reshape(_N, 1)
    m_new_on_ground = gnf
    m_new_on_sub = 1.0 - gnf
    names = ("ground", "g2s", "sub", "s2g")
    masks = (m_new_on_ground, m_new_on_sub, m_new_on_sub, m_new_on_ground)

    we, be = params["embed"]
    wh, bh = params["head"]
    convs = []
    for depth in range(2):
        for si, nm in enumerate(names):
            w, bb = params[nm][depth]
            convs.append((w, bb.reshape(1, _H), si, masks[si]))

    batch2 = batch.reshape(_N, 1)

    h, gl, gr = _embed_tc(x, we, be.reshape(1, _H), convs[0][0],
                          degc[convs[0][2]])

    for k in range(8):
        w_k, b_k, si, m_k = convs[k]
        al, ar = _conv_sc(gl, gr, src1[si], dst3[si])
        if k < 7:
            w_n, _, si_n, _ = convs[k + 1]
            h, gl, gr = _merge_tc(al, ar, h, degc[si], b_k, m_k,
                                  w_n, degc[si_n])
        else:
            out = _final_tc(al, ar, h, degc[si], b_k, m_k, batch2,
                            wh, bh.reshape(1, _OUT))
    return out


# speedup vs baseline: 13.6312x; 1.0974x over previous
import functools

import jax
import jax.numpy as jnp
from jax import lax
from jax.experimental import pallas as pl
from jax.experimental.pallas import tpu as pltpu
from jax.experimental.pallas import tpu_sc as plsc

_N = 10000
_E = 160000
_D = 256
_H = 256
_HH = 128
_OUT = 128
_G = 64
_NS = 16
_CH = 80
_NCH = _E // _NS // _CH
_EPS = _E // _NS
_RPS = 624
_RPS_LAST = _N - 15 * _RPS
_BLK = 1000
_NBLK = _N // _BLK

_f32 = jnp.float32


def _sc_mesh():
    return plsc.VectorSubcoreMesh(core_axis_name="c", subcore_axis_name="s")


def _deg_sc(zeros_n, d0, d1, d2, d3):
    @functools.partial(
        pl.kernel,
        out_type=[jax.ShapeDtypeStruct((_N,), _f32) for _ in range(4)],
        mesh=_sc_mesh(),
        scratch_types=[
            pltpu.VMEM_SHARED((_N,), _f32),
            pltpu.VMEM_SHARED((_N,), _f32),
            pltpu.VMEM((_NCH, _CH), jnp.int32),
            pltpu.VMEM((_CH,), _f32),
        ],
    )
    def k(z_hbm, d0_hbm, d1_hbm, d2_hbm, d3_hbm,
          o0_hbm, o1_hbm, o2_hbm, o3_hbm,
          degA_sh, degB_sh, slab_v, ones_v):
        cid = lax.axis_index("c")
        sid = lax.axis_index("s")

        @pl.when(sid == 0)
        def _():
            pltpu.sync_copy(z_hbm, degA_sh)

        @pl.when(sid == 1)
        def _():
            pltpu.sync_copy(z_hbm, degB_sh)

        for i in range(_CH // 16):
            ones_v[pl.ds(i * 16, 16)] = jnp.ones((16,), _f32)
        plsc.subcore_barrier()

        dsts = (d0_hbm, d1_hbm, d2_hbm, d3_hbm)
        outs = (o0_hbm, o1_hbm, o2_hbm, o3_hbm)
        for c in range(2):
            @pl.when(cid == c)
            def _(c=c):
                for local, acc_sh in ((0, degA_sh), (1, degB_sh)):
                    pltpu.sync_copy(dsts[2 * c + local].at[sid], slab_v)

                    def body(j, _, acc_sh=acc_sh):
                        pltpu.sync_copy(ones_v, acc_sh.at[slab_v.at[j]],
                                        add=True)
                        return 0

                    lax.fori_loop(0, _NCH, body, 0)
        plsc.subcore_barrier()

        for c in range(2):
            @pl.when(cid == c)
            def _(c=c):
                @pl.when(sid == 0)
                def _():
                    pltpu.sync_copy(degA_sh, outs[2 * c])

                @pl.when(sid == 1)
                def _():
                    pltpu.sync_copy(degB_sh, outs[2 * c + 1])

    return k(zeros_n, d0, d1, d2, d3)


@functools.partial(
    pl.kernel,
    out_type=[jax.ShapeDtypeStruct((_N, _HH), _f32),
              jax.ShapeDtypeStruct((_N, _HH), _f32)],
    mesh=_sc_mesh(),
    scratch_types=[
        pltpu.VMEM_SHARED((_N, _HH), _f32),
        pltpu.VMEM((_NCH, _CH), jnp.int32),
        pltpu.VMEM((_CH,), jnp.int32),
        pltpu.VMEM((_CH,), jnp.int32),
        pltpu.VMEM((_CH, _HH), _f32),
        pltpu.VMEM((_CH, _HH), _f32),
        pltpu.SemaphoreType.DMA,
        pltpu.SemaphoreType.DMA,
        pltpu.SemaphoreType.DMA,
        pltpu.SemaphoreType.DMA,
        pltpu.SemaphoreType.DMA,
        pltpu.SemaphoreType.DMA,
    ],
)
def _conv_sc(gl_hbm, gr_hbm, src_hbm, dst_hbm, ol_hbm, or_hbm,
             acc_sh, dst_v, src0_v, src1_v, rows0_v, rows1_v,
             semi0, semi1, semg0, semg1, sems0, sems1):
    cid = lax.axis_index("c")
    sid = lax.axis_index("s")

    pltpu.sync_copy(dst_hbm.at[sid], dst_v)
    ebase = sid * _EPS

    for c, g_hbm, o_hbm in ((0, gl_hbm, ol_hbm), (1, gr_hbm, or_hbm)):
        @pl.when(cid == c)
        def _(g_hbm=g_hbm, o_hbm=o_hbm):
            @pl.when(sid < 15)
            def _():
                pltpu.sync_copy(g_hbm.at[pl.ds(sid * _RPS, _RPS)],
                                acc_sh.at[pl.ds(sid * _RPS, _RPS)])

            @pl.when(sid == 15)
            def _():
                pltpu.sync_copy(g_hbm.at[pl.ds(15 * _RPS, _RPS_LAST)],
                                acc_sh.at[pl.ds(15 * _RPS, _RPS_LAST)])
            plsc.subcore_barrier()

            def idx_cp(j, buf, sem):
                return pltpu.async_copy(
                    src_hbm.at[pl.ds(ebase + j * _CH, _CH)], buf, sem)

            def gat(buf, rows, sem):
                return pltpu.async_copy(g_hbm.at[buf], rows, sem)

            def sca(j, rows, sem):
                return pltpu.async_copy(rows, acc_sh.at[dst_v.at[j]], sem,
                                        add=True)

            idx_cp(0, src0_v, semi0).wait()
            idx_cp(1, src1_v, semi1)
            gat(src0_v, rows0_v, semg0)
            pltpu.make_async_copy(src_hbm.at[pl.ds(ebase, _CH)], src1_v,
                                  semi1).wait()
            gat(src1_v, rows1_v, semg1)

            def pair(i, _):
                j0 = 2 * i
                j1 = 2 * i + 1
                pltpu.make_async_copy(g_hbm.at[src0_v], rows0_v, semg0).wait()
                idx_cp(j0 + 2, src0_v, semi0)
                sca(j0, rows0_v, sems0)
                pltpu.make_async_copy(g_hbm.at[src1_v], rows1_v, semg1).wait()
                idx_cp(j1 + 2, src1_v, semi1)
                sca(j1, rows1_v, sems1)
                pltpu.make_async_copy(rows0_v, acc_sh.at[dst_v.at[j0]],
                                      sems0).wait()
                pltpu.make_async_copy(src_hbm.at[pl.ds(ebase, _CH)], src0_v,
                                      semi0).wait()
                gat(src0_v, rows0_v, semg0)
                pltpu.make_async_copy(rows1_v, acc_sh.at[dst_v.at[j1]],
                                      sems1).wait()
                pltpu.make_async_copy(src_hbm.at[pl.ds(ebase, _CH)], src1_v,
                                      semi1).wait()
                gat(src1_v, rows1_v, semg1)
                return 0

            lax.fori_loop(0, (_NCH - 3) // 2, pair, 0)
            j0 = _NCH - 3
            j1 = _NCH - 2
            jt = _NCH - 1
            pltpu.make_async_copy(g_hbm.at[src0_v], rows0_v, semg0).wait()
            sca(j0, rows0_v, sems0)
            pltpu.make_async_copy(g_hbm.at[src1_v], rows1_v, semg1).wait()
            sca(j1, rows1_v, sems1)
            pltpu.make_async_copy(rows0_v, acc_sh.at[dst_v.at[j0]],
                                  sems0).wait()
            idx_cp(jt, src0_v, semi0).wait()
            gat(src0_v, rows0_v, semg0).wait()
            sca(jt, rows0_v, sems0)
            pltpu.make_async_copy(rows1_v, acc_sh.at[dst_v.at[j1]],
                                  sems1).wait()
            pltpu.make_async_copy(rows0_v, acc_sh.at[dst_v.at[jt]],
                                  sems0).wait()

            plsc.subcore_barrier()

            @pl.when(sid < 15)
            def _():
                pltpu.sync_copy(acc_sh.at[pl.ds(sid * _RPS, _RPS)],
                                o_hbm.at[pl.ds(sid * _RPS, _RPS)])

            @pl.when(sid == 15)
            def _():
                pltpu.sync_copy(acc_sh.at[pl.ds(15 * _RPS, _RPS_LAST)],
                                o_hbm.at[pl.ds(15 * _RPS, _RPS_LAST)])


def _dot(a, b):
    return jnp.dot(a, b, preferred_element_type=_f32)


def _embed_tc(x, we, be, w1, deg1):
    def body(x_ref, we_ref, be_ref, w1_ref, deg_ref, h0_ref, gl_ref, gr_ref):
        h0 = _dot(x_ref[...], we_ref[...]) + be_ref[...]
        h0_ref[...] = h0
        dinv = lax.rsqrt(deg_ref[...] + 1.0)
        g = _dot(h0, w1_ref[...]) * dinv
        gl_ref[...] = g[:, :_HH]
        gr_ref[...] = g[:, _HH:]

    return pl.pallas_call(
        body,
        grid=(_NBLK,),
        in_specs=[
            pl.BlockSpec((_BLK, _D), lambda i: (i, 0)),
            pl.BlockSpec((_D, _H), lambda i: (0, 0)),
            pl.BlockSpec((1, _H), lambda i: (0, 0)),
            pl.BlockSpec((_H, _H), lambda i: (0, 0)),
            pl.BlockSpec((_BLK, 1), lambda i: (i, 0)),
        ],
        out_specs=[
            pl.BlockSpec((_BLK, _H), lambda i: (i, 0)),
            pl.BlockSpec((_BLK, _HH), lambda i: (i, 0)),
            pl.BlockSpec((_BLK, _HH), lambda i: (i, 0)),
        ],
        out_shape=[
            jax.ShapeDtypeStruct((_N, _H), _f32),
            jax.ShapeDtypeStruct((_N, _HH), _f32),
            jax.ShapeDtypeStruct((_N, _HH), _f32),
        ],
    )(x, we, be, w1, deg1)


def _merge_tc(al, ar, h, deg, b, m, wn, degn):
    def body(al_ref, ar_ref, h_ref, deg_ref, b_ref, m_ref, wn_ref, degn_ref,
             ho_ref, gl_ref, gr_ref):
        dinv = lax.rsqrt(deg_ref[...] + 1.0)
        acc = jnp.concatenate([al_ref[...], ar_ref[...]], axis=-1) * dinv
        hn = jnp.where(m_ref[...] == 1.0,
                       jnp.maximum(acc + b_ref[...], 0.0), h_ref[...])
        ho_ref[...] = hn
        g = _dot(hn, wn_ref[...]) * lax.rsqrt(degn_ref[...] + 1.0)
        gl_ref[...] = g[:, :_HH]
        gr_ref[...] = g[:, _HH:]

    return pl.pallas_call(
        body,
        grid=(_NBLK,),
        in_specs=[
            pl.BlockSpec((_BLK, _HH), lambda i: (i, 0)),
            pl.BlockSpec((_BLK, _HH), lambda i: (i, 0)),
            pl.BlockSpec((_BLK, _H), lambda i: (i, 0)),
            pl.BlockSpec((_BLK, 1), lambda i: (i, 0)),
            pl.BlockSpec((1, _H), lambda i: (0, 0)),
            pl.BlockSpec((_BLK, 1), lambda i: (i, 0)),
            pl.BlockSpec((_H, _H), lambda i: (0, 0)),
            pl.BlockSpec((_BLK, 1), lambda i: (i, 0)),
        ],
        out_specs=[
            pl.BlockSpec((_BLK, _H), lambda i: (i, 0)),
            pl.BlockSpec((_BLK, _HH), lambda i: (i, 0)),
            pl.BlockSpec((_BLK, _HH), lambda i: (i, 0)),
        ],
        out_shape=[
            jax.ShapeDtypeStruct((_N, _H), _f32),
            jax.ShapeDtypeStruct((_N, _HH), _f32),
            jax.ShapeDtypeStruct((_N, _HH), _f32),
        ],
    )(al, ar, h, deg, b, m, wn, degn)


def _final_tc(al, ar, h, deg, b, m, batch2, wh, bh):
    def body(al_ref, ar_ref, h_ref, deg_ref, b_ref, m_ref, batch_ref,
             wh_ref, bh_ref, out_ref, pooled):
        i = pl.program_id(0)
        dinv = lax.rsqrt(deg_ref[...] + 1.0)
        acc = jnp.concatenate([al_ref[...], ar_ref[...]], axis=-1) * dinv
        hn = jnp.where(m_ref[...] == 1.0,
                       jnp.maximum(acc + b_ref[...], 0.0), h_ref[...])
        seg = lax.broadcasted_iota(jnp.int32, (1, _G), 1)
        onehot = (batch_ref[...] == seg).astype(_f32)
        contrib = lax.dot_general(onehot, hn, (((0,), (0,)), ((), ())),
                                  preferred_element_type=_f32)

        @pl.when(i == 0)
        def _():
            pooled[...] = contrib

        @pl.when(i > 0)
        def _():
            pooled[...] += contrib

        @pl.when(i == _NBLK - 1)
        def _():
            out_ref[...] = _dot(pooled[...], wh_ref[...]) + bh_ref[...]

    return pl.pallas_call(
        body,
        grid=(_NBLK,),
        in_specs=[
            pl.BlockSpec((_BLK, _HH), lambda i: (i, 0)),
            pl.BlockSpec((_BLK, _HH), lambda i: (i, 0)),
            pl.BlockSpec((_BLK, _H), lambda i: (i, 0)),
            pl.BlockSpec((_BLK, 1), lambda i: (i, 0)),
            pl.BlockSpec((1, _H), lambda i: (0, 0)),
            pl.BlockSpec((_BLK, 1), lambda i: (i, 0)),
            pl.BlockSpec((_BLK, 1), lambda i: (i, 0)),
            pl.BlockSpec((_H, _OUT), lambda i: (0, 0)),
            pl.BlockSpec((1, _OUT), lambda i: (0, 0)),
        ],
        out_specs=pl.BlockSpec((_G, _OUT), lambda i: (0, 0)),
        out_shape=jax.ShapeDtypeStruct((_G, _OUT), _f32),
        scratch_shapes=[pltpu.VMEM((_G, _H), _f32)],
        compiler_params=pltpu.CompilerParams(
            dimension_semantics=("arbitrary",)),
    )(al, ar, h, deg, b, m, batch2, wh, bh)


def kernel(x, edge_index, edge_attr, ground_node, node_subnode_index,
           subgraph_edge_index, subnode_node_index, batch, params):
    sets = (edge_index, node_subnode_index, subgraph_edge_index,
            subnode_node_index)
    src1 = [s[0] for s in sets]
    dst3 = [s[1].reshape(_NS, _NCH, _CH) for s in sets]

    degs = _deg_sc(jnp.zeros((_N,), _f32),
                   dst3[0], dst3[1], dst3[2], dst3[3])
    degc = [d.reshape(_N, 1) for d in degs]

    gnf = ground_node.astype(_f32).reshape(_N, 1)
    m_new_on_ground = gnf
    m_new_on_sub = 1.0 - gnf
    names = ("ground", "g2s", "sub", "s2g")
    masks = (m_new_on_ground, m_new_on_sub, m_new_on_sub, m_new_on_ground)

    we, be = params["embed"]
    wh, bh = params["head"]
    convs = []
    for depth in range(2):
        for si, nm in enumerate(names):
            w, bb = params[nm][depth]
            convs.append((w, bb.reshape(1, _H), si, masks[si]))

    batch2 = batch.reshape(_N, 1)

    h, gl, gr = _embed_tc(x, we, be.reshape(1, _H), convs[0][0],
                          degc[convs[0][2]])

    for k in range(8):
        w_k, b_k, si, m_k = convs[k]
        al, ar = _conv_sc(gl, gr, src1[si], dst3[si])
        if k < 7:
            w_n, _, si_n, _ = convs[k + 1]
            h, gl, gr = _merge_tc(al, ar, h, degc[si], b_k, m_k,
                                  w_n, degc[si_n])
        else:
            out = _final_tc(al, ar, h, degc[si], b_k, m_k, batch2,
                            wh, bh.reshape(1, _OUT))
    return out
